# baseline (speedup 1.0000x reference)
import jax
import jax.numpy as jnp
from jax import lax
from jax.experimental import pallas as pl
from jax.experimental.pallas import tpu as pltpu
from jax.experimental.pallas import tpu_sc as plsc

_NUM_CLASSES = 100000
_FEAT_DIM = 64
_BATCH = 16384

_NC = 2
_NS = 16
_NW = _NC * _NS
_DPW = _FEAT_DIM // _NW
_SUB = 2048
_NSUB = _BATCH // _SUB
_L = 16


def _center_loss_body(labels_hbm, ft_hbm, ct_hbm, out_hbm,
                      crow_v, lab_v, frow_v, acc_v, csem, fsem, lsem):
    wid = lax.axis_index("s") * _NC + lax.axis_index("c")

    lcopy = pltpu.async_copy(labels_hbm.at[...], lab_v, lsem)

    accs = tuple(jnp.zeros((_L,), jnp.float32) for _ in range(8))
    for u in range(_DPW):
        d = wid * _DPW + u
        ccopy = pltpu.async_copy(ct_hbm.at[d, :], crow_v, csem)

        def fcopy(ch):
            return pltpu.async_copy(
                ft_hbm.at[d, pl.ds(ch * _SUB, _SUB)], frow_v.at[ch % 2], fsem)

        pending = [fcopy(0), fcopy(1)]
        if u == 0:
            lcopy.wait()
        ccopy.wait()
        for ch in range(_NSUB):
            pending[ch].wait()

            def grp_body(g, accs):
                lv = lab_v[pl.ds(ch * _SUB + g * _L, _L)]
                cv = plsc.load_gather(crow_v, [lv])
                fv = frow_v[ch % 2, pl.ds(g * _L, _L)]
                dd = fv - cv
                return accs[1:] + (accs[0] + dd * dd,)

            accs = lax.fori_loop(0, _SUB // _L, grp_body, accs, unroll=8)
            if ch + 2 < _NSUB:
                pending.append(fcopy(ch + 2))

    acc = accs[0]
    for a in accs[1:]:
        acc = acc + a
    acc_v[...] = acc
    pltpu.sync_copy(acc_v, out_hbm.at[wid])


@jax.jit
def _center_loss_sc(labels, features, centers):
    ct = centers.T
    ft = features.T
    mesh = plsc.VectorSubcoreMesh(core_axis_name="c", subcore_axis_name="s")
    partials = pl.kernel(
        _center_loss_body,
        mesh=mesh,
        compiler_params=pltpu.CompilerParams(needs_layout_passes=False),
        out_type=jax.ShapeDtypeStruct((_NW, _L), jnp.float32),
        scratch_types=[
            pltpu.VMEM((_NUM_CLASSES,), jnp.float32),
            pltpu.VMEM((_BATCH,), jnp.int32),
            pltpu.VMEM((2, _SUB), jnp.float32),
            pltpu.VMEM((_L,), jnp.float32),
            pltpu.SemaphoreType.DMA,
            pltpu.SemaphoreType.DMA,
            pltpu.SemaphoreType.DMA,
        ],
    )(labels, ft, ct)
    return jnp.sum(partials) * (0.5 / _BATCH)


def kernel(features, labels, centers):
    return _center_loss_sc(labels.astype(jnp.int32), features, centers)


# speedup vs baseline: 1.0852x; 1.0852x over previous
import jax
import jax.numpy as jnp
from jax import lax
from jax.experimental import pallas as pl
from jax.experimental.pallas import tpu as pltpu
from jax.experimental.pallas import tpu_sc as plsc

_NUM_CLASSES = 100000
_FEAT_DIM = 64
_BATCH = 16384

_NC = 2
_NS = 16
_NW = _NC * _NS
_DPW = _FEAT_DIM // _NW
_SUB = 4096
_NSUB = _BATCH // _SUB
_L = 16


def _center_loss_body(labels_hbm, ft_hbm, ct_hbm, out_hbm,
                      crow_v, lab_v, frow_v, acc_v, csem, fsem, lsem):
    wid = lax.axis_index("s") * _NC + lax.axis_index("c")

    lcopy = pltpu.async_copy(labels_hbm.at[...], lab_v, lsem)

    accs = tuple(jnp.zeros((_L,), jnp.float32) for _ in range(8))
    for u in range(_DPW):
        d = wid * _DPW + u
        ccopy = pltpu.async_copy(ct_hbm.at[d, :], crow_v, csem)

        def fcopy(ch):
            return pltpu.async_copy(
                ft_hbm.at[d, pl.ds(ch * _SUB, _SUB)], frow_v.at[ch % 2], fsem)

        pending = [fcopy(0), fcopy(1)]
        if u == 0:
            lcopy.wait()
        ccopy.wait()
        for ch in range(_NSUB):
            pending[ch].wait()

            def grp_body(g, accs):
                lv = lab_v[pl.ds(ch * _SUB + g * _L, _L)]
                cv = plsc.load_gather(crow_v, [lv])
                fv = frow_v[ch % 2, pl.ds(g * _L, _L)]
                dd = fv - cv
                return accs[1:] + (accs[0] + dd * dd,)

            accs = lax.fori_loop(0, _SUB // _L, grp_body, accs, unroll=8)
            if ch + 2 < _NSUB:
                pending.append(fcopy(ch + 2))

    acc = accs[0]
    for a in accs[1:]:
        acc = acc + a
    acc_v[...] = acc
    pltpu.sync_copy(acc_v, out_hbm.at[wid])


@jax.jit
def _center_loss_sc(labels, features, centers):
    ct = centers.T
    ft = features.T
    mesh = plsc.VectorSubcoreMesh(core_axis_name="c", subcore_axis_name="s")
    partials = pl.kernel(
        _center_loss_body,
        mesh=mesh,
        compiler_params=pltpu.CompilerParams(needs_layout_passes=False),
        out_type=jax.ShapeDtypeStruct((_NW, _L), jnp.float32),
        scratch_types=[
            pltpu.VMEM((_NUM_CLASSES,), jnp.float32),
            pltpu.VMEM((_BATCH,), jnp.int32),
            pltpu.VMEM((2, _SUB), jnp.float32),
            pltpu.VMEM((_L,), jnp.float32),
            pltpu.SemaphoreType.DMA,
            pltpu.SemaphoreType.DMA,
            pltpu.SemaphoreType.DMA,
        ],
    )(labels, ft, ct)
    return jnp.sum(partials) * (0.5 / _BATCH)


def kernel(features, labels, centers):
    return _center_loss_sc(labels.astype(jnp.int32), features, centers)
